# baseline (speedup 1.0000x reference)
import functools

import jax
import jax.numpy as jnp
from jax import lax
from jax.experimental import pallas as pl
from jax.experimental.pallas import tpu as pltpu
from jax.experimental.pallas import tpu_sc as plsc

N = 10000
NE = 330000
NC, NS = 2, 16
NW = NC * NS
RT = N // NS
NH = 10008
BR = 1000
G = N // BR
GP = G + 1

_GEOM = {32: (128, 81), 64: (128, 81), 128: (80, 129)}

_f32 = jnp.float32


def _hext(h, a_s, a_d, pad):
    es = jnp.sum(h * a_s, axis=1, keepdims=True)
    ed3 = jnp.sum(h * a_d, axis=1).reshape(1, 1, BR)
    es = jnp.where(pad, -1e30, es)
    h = jnp.where(pad, 0.0, h)
    hx = jnp.concatenate([h, es, jnp.zeros((BR, 15), _f32)], axis=1)
    return hx, ed3


def _pad_mask():
    row = (lax.broadcasted_iota(jnp.int32, (BR, 1), 0)
           + pl.program_id(0) * BR)
    return row >= N


def _tc_first_body(x_ref, w_ref, as_ref, ad_ref, h_ref, ed_ref):
    h = jnp.dot(x_ref[...], w_ref[...], preferred_element_type=_f32)
    h_ref[...], ed_ref[...] = _hext(h, as_ref[...], ad_ref[...], _pad_mask())


def _norm_x(acc_ref, b_ref, din):
    s = acc_ref[0] + acc_ref[1]
    den = s[:, din:din + 1]
    return jnp.maximum(s[:, :din] / den + b_ref[...], 0.0)


def _tc_mid_body(din, acc_ref, b_ref, w_ref, as_ref, ad_ref, h_ref, ed_ref):
    x = _norm_x(acc_ref, b_ref, din)
    h = jnp.dot(x, w_ref[...], preferred_element_type=_f32)
    h_ref[...], ed_ref[...] = _hext(h, as_ref[...], ad_ref[...], _pad_mask())


def _tc_final_body(acc_ref, b_ref, wfc_ref, bfc_ref, out_ref):
    x = _norm_x(acc_ref, b_ref, 128)
    out_ref[...] = (jnp.dot(x, wfc_ref[...], preferred_element_type=_f32)
                    + bfc_ref[...])


def _whole(shape):
    return pl.BlockSpec(shape, lambda i: (0,) * len(shape))


def _clip(i):
    return jnp.minimum(i, G - 1)


def _tc_first(x, w, a_s, a_d):
    din, dout = w.shape
    return pl.pallas_call(
        _tc_first_body,
        grid=(GP,),
        in_specs=[
            pl.BlockSpec((BR, din), lambda i: (_clip(i), 0)),
            _whole((din, dout)), _whole((1, dout)), _whole((1, dout)),
        ],
        out_specs=[
            pl.BlockSpec((BR, dout + 16), lambda i: (i, 0)),
            pl.BlockSpec((1, 1, BR), lambda i: (_clip(i), 0, 0)),
        ],
        out_shape=[
            jax.ShapeDtypeStruct((NH, dout + 16), _f32),
            jax.ShapeDtypeStruct((G, 1, BR), _f32),
        ],
    )(x, w, a_s, a_d)


def _tc_mid(accp, b, w, a_s, a_d):
    din, dout = w.shape
    dc = din + 16
    return pl.pallas_call(
        functools.partial(_tc_mid_body, din),
        grid=(GP,),
        in_specs=[
            pl.BlockSpec((NC, BR, dc), lambda i: (0, _clip(i), 0)),
            _whole((1, din)), _whole((din, dout)),
            _whole((1, dout)), _whole((1, dout)),
        ],
        out_specs=[
            pl.BlockSpec((BR, dout + 16), lambda i: (i, 0)),
            pl.BlockSpec((1, 1, BR), lambda i: (_clip(i), 0, 0)),
        ],
        out_shape=[
            jax.ShapeDtypeStruct((NH, dout + 16), _f32),
            jax.ShapeDtypeStruct((G, 1, BR), _f32),
        ],
    )(accp, b, w, a_s, a_d)


def _tc_final(accp, b, wfc, bfc):
    dc = 128 + 16
    return pl.pallas_call(
        _tc_final_body,
        grid=(G,),
        in_specs=[
            pl.BlockSpec((NC, BR, dc), lambda i: (0, i, 0)),
            _whole((1, 128)), _whole((128, 128)), _whole((1, 128)),
        ],
        out_specs=pl.BlockSpec((BR, 128), lambda i: (i, 0)),
        out_shape=jax.ShapeDtypeStruct((N, 128), _f32),
    )(accp, b, wfc, bfc)


@functools.lru_cache(maxsize=None)
def _sc_edge(dout):
    dc = dout + 16
    cg = dout // 16
    chunk, nch = _GEOM[dout]
    full, rem = RT // chunk, RT % chunk
    mesh = plsc.VectorSubcoreMesh(core_axis_name="c", subcore_axis_name="s",
                                  num_cores=NC, num_subcores=NS)

    @functools.partial(
        pl.kernel,
        out_type=jax.ShapeDtypeStruct((NC, N, dc), _f32),
        mesh=mesh,
        compiler_params=pltpu.CompilerParams(needs_layout_passes=False,
                                             use_tc_tiling_on_sc=False),
        scratch_types=[
            pltpu.VMEM((4, 2, chunk), jnp.int32),
            pltpu.VMEM((3, chunk), _f32),
            pltpu.VMEM((chunk,), _f32),
            pltpu.VMEM((3, chunk, dc), _f32),
            pltpu.VMEM_SHARED((N, dc), _f32),
            pltpu.SemaphoreType.DMA,
            pltpu.SemaphoreType.DMA,
        ],
    )
    def sc_fn(ed_hbm, sd_hbm, h_hbm, out_hbm,
              sdv, edg, wv, land, acc, gsem, ssem):
        cid = lax.axis_index("c")
        sid = lax.axis_index("s")
        wid = cid * NS + sid

        def _zero_row(r, _):
            for g in range(dc // 16):
                land[0, r, pl.ds(g * 16, 16)] = jnp.zeros((16,), _f32)
            return 0
        lax.fori_loop(0, chunk, _zero_row, 0)
        for k in range(full):
            pltpu.sync_copy(land.at[0],
                            acc.at[pl.ds(sid * RT + k * chunk, chunk)])
        if rem:
            pltpu.sync_copy(land.at[0, pl.ds(0, rem)],
                            acc.at[pl.ds(sid * RT + full * chunk, rem)])
        plsc.subcore_barrier()

        lanes = lax.iota(jnp.int32, 16)
        col_es = jnp.full((16,), dout, jnp.int32)
        pltpu.sync_copy(sd_hbm.at[wid * nch], sdv.at[0])
        pltpu.sync_copy(sd_hbm.at[wid * nch + 1], sdv.at[1])
        pltpu.async_copy(h_hbm.at[sdv.at[0, 0]], land.at[0], gsem)
        pltpu.async_copy(ed_hbm.at[sdv.at[0, 1]], edg.at[0], gsem)

        def _iter(j, _):
            b = lax.rem(j, 3)
            bn = lax.rem(j + 1, 3)
            q = lax.rem(j, 4)
            qn = lax.rem(j + 1, 4)
            q2 = lax.rem(j + 2, 4)

            @pl.when(j >= 2)
            def _drain_scatter():
                pltpu.make_async_copy(
                    h_hbm.at[pl.ds(0, chunk)], land.at[0], ssem).wait()

            @pl.when(jnp.logical_and(j >= 1, j < nch - 1))
            def _drain_idx():
                pltpu.make_async_copy(sd_hbm.at[0], sdv.at[0], gsem).wait()

            @pl.when(j < nch - 2)
            def _prefetch_idx():
                pltpu.async_copy(sd_hbm.at[wid * nch + j + 2],
                                 sdv.at[q2], gsem)

            @pl.when(j < nch - 1)
            def _issue():
                pltpu.async_copy(h_hbm.at[sdv.at[qn, 0]], land.at[bn], gsem)
                pltpu.async_copy(ed_hbm.at[sdv.at[qn, 1]], edg.at[bn], gsem)

            @pl.when(j < nch)
            def _compute():
                pltpu.make_async_copy(
                    h_hbm.at[pl.ds(0, chunk)], land.at[0], gsem).wait()
                pltpu.make_async_copy(
                    ed_hbm.at[pl.ds(0, chunk)], edg.at[0], gsem).wait()

                @plsc.parallel_loop(0, chunk // 16, unroll=2)
                def _wgrp(g):
                    e = (plsc.load_gather(land.at[b],
                                          [lanes + g * 16, col_es])
                         + edg[b, pl.ds(g * 16, 16)])
                    wv[pl.ds(g * 16, 16)] = jnp.exp(jnp.maximum(e, 0.2 * e))

                @plsc.parallel_loop(0, chunk, unroll=8)
                def _edge(i):
                    wbc = plsc.load_gather(
                        wv, [jnp.full((16,), i, jnp.int32)])
                    for g in range(cg):
                        land[b, i, pl.ds(g * 16, 16)] = (
                            land[b, i, pl.ds(g * 16, 16)] * wbc)
                    land[b, i, pl.ds(dout, 16)] = wbc

                pltpu.async_copy(land.at[b], acc.at[sdv.at[q, 1]],
                                 ssem, add=True)
            return 0
        lax.fori_loop(0, nch + 2, _iter, 0)

        plsc.subcore_barrier()
        pltpu.sync_copy(acc.at[pl.ds(sid * RT, RT)],
                        out_hbm.at[cid, pl.ds(sid * RT, RT)])

    return sc_fn


@functools.lru_cache(maxsize=None)
def _sd_geom(dout):
    chunk, nch = _GEOM[dout]
    return chunk, nch, NW * nch * chunk


def kernel(x, edge_index, W1, as1, ad1, b1, W2, as2, ad2, b2,
           W3, as3, ad3, b3, W4, as4, ad4, b4, Wfc, bfc):
    loops = jnp.arange(N, dtype=jnp.int32)
    sds = {}
    for dout in (32, 128):
        chunk, nch, ep = _sd_geom(dout)
        src = jnp.concatenate(
            [edge_index[0], loops, jnp.full((ep - NE,), N, jnp.int32)])
        dst = jnp.concatenate(
            [edge_index[1], loops, jnp.zeros((ep - NE,), jnp.int32)])
        sds[dout] = jnp.stack([src.reshape(NW * nch, chunk),
                               dst.reshape(NW * nch, chunk)], axis=1)
    sds[64] = sds[32]

    r2 = lambda a: a.reshape(1, -1)
    h, ed3 = _tc_first(x, W1, r2(as1), r2(ad1))
    accp = _sc_edge(W1.shape[1])(ed3.reshape(N), sds[W1.shape[1]], h)
    for (W, a_s, a_d, bprev) in ((W2, as2, ad2, b1), (W3, as3, ad3, b2),
                                 (W4, as4, ad4, b3)):
        h, ed3 = _tc_mid(accp, r2(bprev), W, r2(a_s), r2(a_d))
        accp = _sc_edge(W.shape[1])(ed3.reshape(N), sds[W.shape[1]], h)
    return _tc_final(accp, r2(b4), Wfc, r2(bfc))


# speedup vs baseline: 1.0479x; 1.0479x over previous
import functools

import jax
import jax.numpy as jnp
from jax import lax
from jax.experimental import pallas as pl
from jax.experimental.pallas import tpu as pltpu
from jax.experimental.pallas import tpu_sc as plsc

N = 10000
NE = 330000
NC, NS = 2, 16
NW = NC * NS
RT = N // NS
NH = 10008
BR = 1000
G = N // BR
GP = G + 1

_GEOM = {32: (128, 81), 64: (128, 81), 128: (80, 129)}

_f32 = jnp.float32


def _hext(h, a_s, a_d, pad):
    es = jnp.sum(h * a_s, axis=1, keepdims=True)
    ed3 = jnp.sum(h * a_d, axis=1).reshape(1, 1, BR)
    es = jnp.where(pad, -1e30, es)
    h = jnp.where(pad, 0.0, h)
    hx = jnp.concatenate([h, es, jnp.zeros((BR, 15), _f32)], axis=1)
    return hx, ed3


def _pad_mask():
    row = (lax.broadcasted_iota(jnp.int32, (BR, 1), 0)
           + pl.program_id(0) * BR)
    return row >= N


def _tc_first_body(x_ref, w_ref, as_ref, ad_ref, h_ref, ed_ref):
    h = jnp.dot(x_ref[...], w_ref[...], preferred_element_type=_f32)
    h_ref[...], ed_ref[...] = _hext(h, as_ref[...], ad_ref[...], _pad_mask())


def _norm_x(acc_ref, b_ref, din):
    s = acc_ref[0] + acc_ref[1]
    den = s[:, din:din + 1]
    return jnp.maximum(s[:, :din] / den + b_ref[...], 0.0)


def _tc_mid_body(din, acc_ref, b_ref, w_ref, as_ref, ad_ref, h_ref, ed_ref):
    x = _norm_x(acc_ref, b_ref, din)
    h = jnp.dot(x, w_ref[...], preferred_element_type=_f32)
    h_ref[...], ed_ref[...] = _hext(h, as_ref[...], ad_ref[...], _pad_mask())


def _tc_final_body(acc_ref, b_ref, wfc_ref, bfc_ref, out_ref):
    x = _norm_x(acc_ref, b_ref, 128)
    out_ref[...] = (jnp.dot(x, wfc_ref[...], preferred_element_type=_f32)
                    + bfc_ref[...])


def _whole(shape):
    return pl.BlockSpec(shape, lambda i: (0,) * len(shape))


def _clip(i):
    return jnp.minimum(i, G - 1)


def _tc_first(x, w, a_s, a_d):
    din, dout = w.shape
    return pl.pallas_call(
        _tc_first_body,
        grid=(GP,),
        in_specs=[
            pl.BlockSpec((BR, din), lambda i: (_clip(i), 0)),
            _whole((din, dout)), _whole((1, dout)), _whole((1, dout)),
        ],
        out_specs=[
            pl.BlockSpec((BR, dout + 16), lambda i: (i, 0)),
            pl.BlockSpec((1, 1, BR), lambda i: (_clip(i), 0, 0)),
        ],
        out_shape=[
            jax.ShapeDtypeStruct((NH, dout + 16), _f32),
            jax.ShapeDtypeStruct((G, 1, BR), _f32),
        ],
    )(x, w, a_s, a_d)


def _tc_mid(accp, b, w, a_s, a_d):
    din, dout = w.shape
    dc = din + 16
    return pl.pallas_call(
        functools.partial(_tc_mid_body, din),
        grid=(GP,),
        in_specs=[
            pl.BlockSpec((NC, BR, dc), lambda i: (0, _clip(i), 0)),
            _whole((1, din)), _whole((din, dout)),
            _whole((1, dout)), _whole((1, dout)),
        ],
        out_specs=[
            pl.BlockSpec((BR, dout + 16), lambda i: (i, 0)),
            pl.BlockSpec((1, 1, BR), lambda i: (_clip(i), 0, 0)),
        ],
        out_shape=[
            jax.ShapeDtypeStruct((NH, dout + 16), _f32),
            jax.ShapeDtypeStruct((G, 1, BR), _f32),
        ],
    )(accp, b, w, a_s, a_d)


def _tc_final(accp, b, wfc, bfc):
    dc = 128 + 16
    return pl.pallas_call(
        _tc_final_body,
        grid=(G,),
        in_specs=[
            pl.BlockSpec((NC, BR, dc), lambda i: (0, i, 0)),
            _whole((1, 128)), _whole((128, 128)), _whole((1, 128)),
        ],
        out_specs=pl.BlockSpec((BR, 128), lambda i: (i, 0)),
        out_shape=jax.ShapeDtypeStruct((N, 128), _f32),
    )(accp, b, wfc, bfc)


@functools.lru_cache(maxsize=None)
def _sc_edge(dout):
    dc = dout + 16
    cg = dout // 16
    chunk, nch = _GEOM[dout]
    full, rem = RT // chunk, RT % chunk
    mesh = plsc.VectorSubcoreMesh(core_axis_name="c", subcore_axis_name="s",
                                  num_cores=NC, num_subcores=NS)

    @functools.partial(
        pl.kernel,
        out_type=jax.ShapeDtypeStruct((NC, N, dc), _f32),
        mesh=mesh,
        compiler_params=pltpu.CompilerParams(needs_layout_passes=False,
                                             use_tc_tiling_on_sc=False),
        scratch_types=[
            pltpu.VMEM((4, 2, chunk), jnp.int32),
            pltpu.VMEM((3, chunk) if dout > 64 else (N,), _f32),
            pltpu.VMEM((chunk,), _f32),
            pltpu.VMEM((3, chunk, dc), _f32),
            pltpu.VMEM_SHARED((N, dc), _f32),
            pltpu.SemaphoreType.DMA,
            pltpu.SemaphoreType.DMA,
        ],
    )
    def sc_fn(ed_hbm, sd_hbm, h_hbm, out_hbm,
              sdv, edg, wv, land, acc, gsem, ssem):
        cid = lax.axis_index("c")
        sid = lax.axis_index("s")
        wid = cid * NS + sid

        def _zero_row(r, _):
            for g in range(dc // 16):
                land[0, r, pl.ds(g * 16, 16)] = jnp.zeros((16,), _f32)
            return 0
        lax.fori_loop(0, chunk, _zero_row, 0)
        for k in range(full):
            pltpu.sync_copy(land.at[0],
                            acc.at[pl.ds(sid * RT + k * chunk, chunk)])
        if rem:
            pltpu.sync_copy(land.at[0, pl.ds(0, rem)],
                            acc.at[pl.ds(sid * RT + full * chunk, rem)])
        if dout <= 64:
            pltpu.sync_copy(ed_hbm, edg)
        plsc.subcore_barrier()

        lanes = lax.iota(jnp.int32, 16)
        col_es = jnp.full((16,), dout, jnp.int32)
        pltpu.sync_copy(sd_hbm.at[wid * nch], sdv.at[0])
        pltpu.sync_copy(sd_hbm.at[wid * nch + 1], sdv.at[1])
        pltpu.async_copy(h_hbm.at[sdv.at[0, 0]], land.at[0], gsem)
        if dout > 64:
            pltpu.async_copy(ed_hbm.at[sdv.at[0, 1]], edg.at[0], gsem)

        def _iter(j, _):
            b = lax.rem(j, 3)
            bn = lax.rem(j + 1, 3)
            q = lax.rem(j, 4)
            qn = lax.rem(j + 1, 4)
            q2 = lax.rem(j + 2, 4)

            @pl.when(j >= 2)
            def _drain_scatter():
                pltpu.make_async_copy(
                    h_hbm.at[pl.ds(0, chunk)], land.at[0], ssem).wait()

            @pl.when(jnp.logical_and(j >= 1, j < nch - 1))
            def _drain_idx():
                pltpu.make_async_copy(sd_hbm.at[0], sdv.at[0], gsem).wait()

            @pl.when(j < nch - 2)
            def _prefetch_idx():
                pltpu.async_copy(sd_hbm.at[wid * nch + j + 2],
                                 sdv.at[q2], gsem)

            @pl.when(j < nch - 1)
            def _issue():
                pltpu.async_copy(h_hbm.at[sdv.at[qn, 0]], land.at[bn], gsem)
                if dout > 64:
                    pltpu.async_copy(ed_hbm.at[sdv.at[qn, 1]],
                                     edg.at[bn], gsem)

            @pl.when(j < nch)
            def _compute():
                pltpu.make_async_copy(
                    h_hbm.at[pl.ds(0, chunk)], land.at[0], gsem).wait()
                if dout > 64:
                    pltpu.make_async_copy(
                        ed_hbm.at[pl.ds(0, chunk)], edg.at[0], gsem).wait()

                @plsc.parallel_loop(0, chunk // 16, unroll=2)
                def _wgrp(g):
                    es16 = plsc.load_gather(land.at[b],
                                            [lanes + g * 16, col_es])
                    if dout > 64:
                        ed16 = edg[b, pl.ds(g * 16, 16)]
                    else:
                        ed16 = plsc.load_gather(
                            edg, [sdv[q, 1, pl.ds(g * 16, 16)]])
                    e = es16 + ed16
                    wv[pl.ds(g * 16, 16)] = jnp.exp(jnp.maximum(e, 0.2 * e))

                @plsc.parallel_loop(0, chunk, unroll=8)
                def _edge(i):
                    wbc = plsc.load_gather(
                        wv, [jnp.full((16,), i, jnp.int32)])
                    for g in range(cg):
                        land[b, i, pl.ds(g * 16, 16)] = (
                            land[b, i, pl.ds(g * 16, 16)] * wbc)
                    land[b, i, pl.ds(dout, 16)] = wbc

                pltpu.async_copy(land.at[b], acc.at[sdv.at[q, 1]],
                                 ssem, add=True)
            return 0
        lax.fori_loop(0, nch + 2, _iter, 0)

        plsc.subcore_barrier()
        pltpu.sync_copy(acc.at[pl.ds(sid * RT, RT)],
                        out_hbm.at[cid, pl.ds(sid * RT, RT)])

    return sc_fn


@functools.lru_cache(maxsize=None)
def _sd_geom(dout):
    chunk, nch = _GEOM[dout]
    return chunk, nch, NW * nch * chunk


def kernel(x, edge_index, W1, as1, ad1, b1, W2, as2, ad2, b2,
           W3, as3, ad3, b3, W4, as4, ad4, b4, Wfc, bfc):
    loops = jnp.arange(N, dtype=jnp.int32)
    sds = {}
    for dout in (32, 128):
        chunk, nch, ep = _sd_geom(dout)
        src = jnp.concatenate(
            [edge_index[0], loops, jnp.full((ep - NE,), N, jnp.int32)])
        dst = jnp.concatenate(
            [edge_index[1], loops, jnp.zeros((ep - NE,), jnp.int32)])
        sds[dout] = jnp.stack([src.reshape(NW * nch, chunk),
                               dst.reshape(NW * nch, chunk)], axis=1)
    sds[64] = sds[32]

    r2 = lambda a: a.reshape(1, -1)
    h, ed3 = _tc_first(x, W1, r2(as1), r2(ad1))
    accp = _sc_edge(W1.shape[1])(ed3.reshape(N), sds[W1.shape[1]], h)
    for (W, a_s, a_d, bprev) in ((W2, as2, ad2, b1), (W3, as3, ad3, b2),
                                 (W4, as4, ad4, b3)):
        h, ed3 = _tc_mid(accp, r2(bprev), W, r2(a_s), r2(a_d))
        accp = _sc_edge(W.shape[1])(ed3.reshape(N), sds[W.shape[1]], h)
    return _tc_final(accp, r2(b4), Wfc, r2(bfc))
